# baseline (speedup 1.0000x reference)
import jax
import jax.numpy as jnp
from jax import lax
from jax.experimental import pallas as pl
from jax.experimental.pallas import tpu as pltpu
from jax.experimental.pallas import tpu_sc as plsc

VOCAB = 1000000
EMB = 64
PHOTO_DIM = 1024
B = 1024
L = 50

R = B * L
NC, NS = 2, 16
NW = NC * NS
ROWS_PER_W = R // NW
CHUNK = 80
NCHUNK = ROWS_PER_W // CHUNK

RB = 400
BB = RB // L


def _sc_gather_body(table_hbm, idx_hbm, out_hbm, idx_v, rows_v, sem):
    wid = lax.axis_index("s") * NC + lax.axis_index("c")
    base = wid * ROWS_PER_W
    pltpu.sync_copy(idx_hbm.at[wid], idx_v)
    copies = []
    for j in range(NCHUNK):
        c = pltpu.make_async_copy(
            table_hbm.at[idx_v.at[j]],
            rows_v.at[pl.ds(j * CHUNK, CHUNK)],
            sem,
        )
        c.start()
        copies.append(c)
    for c in copies:
        c.wait()
    pltpu.sync_copy(rows_v, out_hbm.at[pl.ds(base, ROWS_PER_W)])


def _sc_gather(word_table, idx3):
    mesh = plsc.VectorSubcoreMesh(core_axis_name="c", subcore_axis_name="s")
    return pl.kernel(
        _sc_gather_body,
        mesh=mesh,
        out_type=jax.ShapeDtypeStruct((R, EMB), jnp.float32),
        scratch_types=[
            pltpu.VMEM((NCHUNK, CHUNK), jnp.int32),
            pltpu.VMEM((ROWS_PER_W, EMB), jnp.float32),
            pltpu.SemaphoreType.DMA,
        ],
        compiler_params=pltpu.CompilerParams(use_tc_tiling_on_sc=False),
    )(word_table, idx3)


def _tc_body(vis_ref, word_ref, seg_ref, pos_ref, segtab_ref, w_ref,
             g_ref, bta_ref, out_ref):
    acc = jnp.dot(vis_ref[...], w_ref[...], preferred_element_type=jnp.float32)
    segc = seg_ref[...]
    t0 = segtab_ref[0:1, :]
    t1 = segtab_ref[1:2, :]
    t2 = segtab_ref[2:3, :]
    seg_e = jnp.where(segc == 0.0, t0, jnp.where(segc == 1.0, t1, t2))
    emb = acc + word_ref[...] + pos_ref[...] + seg_e
    mean = jnp.mean(emb, axis=1, keepdims=True)
    d = emb - mean
    var = jnp.mean(d * d, axis=1, keepdims=True)
    out_ref[...] = d * lax.rsqrt(var + 1e-6) * g_ref[...] + bta_ref[...]


def _tc_fused(vis2, word_emb, seg_f, pos_block, seg_table, w_vis, g2, b2,
              interpret=False):
    grid = (R // RB,)
    return pl.pallas_call(
        _tc_body,
        grid=grid,
        in_specs=[
            pl.BlockSpec((RB, PHOTO_DIM), lambda i: (i, 0)),
            pl.BlockSpec((RB, EMB), lambda i: (i, 0)),
            pl.BlockSpec((RB, 1), lambda i: (i, 0)),
            pl.BlockSpec((RB, EMB), lambda i: (0, 0)),
            pl.BlockSpec((3, EMB), lambda i: (0, 0)),
            pl.BlockSpec((PHOTO_DIM, EMB), lambda i: (0, 0)),
            pl.BlockSpec((1, EMB), lambda i: (0, 0)),
            pl.BlockSpec((1, EMB), lambda i: (0, 0)),
        ],
        out_specs=pl.BlockSpec((RB, EMB), lambda i: (i, 0)),
        out_shape=jax.ShapeDtypeStruct((R, EMB), jnp.float32),
        compiler_params=pltpu.CompilerParams(
            dimension_semantics=("arbitrary",),
        ),
        interpret=interpret,
    )(vis2, word_emb, seg_f, pos_block, seg_table, w_vis, g2, b2)


def kernel(visual, src, seg, word_table, pos_table, seg_table, W_vis,
           b_vis, ln_gamma, ln_beta):
    idx3 = src.reshape(NW, NCHUNK, CHUNK)
    word_emb = _sc_gather(word_table, idx3)
    vis2 = visual.reshape(R, PHOTO_DIM)
    seg_f = seg.reshape(R, 1).astype(jnp.float32)
    pos_block = jnp.tile(pos_table[:L] + b_vis, (BB, 1))
    g2 = ln_gamma.reshape(1, EMB)
    b2 = ln_beta.reshape(1, EMB)
    out2 = _tc_fused(vis2, word_emb, seg_f, pos_block, seg_table, W_vis,
                     g2, b2)
    return out2.reshape(B, L, EMB)


# speedup vs baseline: 2.4501x; 2.4501x over previous
import jax
import jax.numpy as jnp
from jax import lax
from jax.experimental import pallas as pl
from jax.experimental.pallas import tpu as pltpu
from jax.experimental.pallas import tpu_sc as plsc

VOCAB = 1000000
EMB = 64
PHOTO_DIM = 1024
B = 1024
L = 50

R = B * L
NC, NS = 2, 16
NW = NC * NS
ROWS_PER_W = R // NW
CHUNK = 32
NCHUNK = ROWS_PER_W // CHUNK

BB = 8
RB = BB * L


def _sc_gather_body(table_hbm, idx_hbm, out_hbm,
                    idx_v, out_v,
                    gsem0, gsem1, wsem0, wsem1):
    wid = lax.axis_index("s") * NC + lax.axis_index("c")
    base = wid * ROWS_PER_W
    pltpu.sync_copy(idx_hbm.at[wid], idx_v)
    gsems = (gsem0, gsem1)
    wsems = (wsem0, wsem1)

    def chunk_pair(g, carry):
        for b in range(2):
            c = 2 * g + b

            @pl.when(g > 0)
            def _():
                pltpu.make_async_copy(
                    out_v.at[b], out_hbm.at[pl.ds(0, CHUNK)],
                    wsems[b]).wait()

            for grp in range(CHUNK // 16):
                vec = idx_v[c, pl.ds(grp * 16, 16)]
                for lane in range(16):
                    i = vec[lane]
                    pltpu.make_async_copy(
                        table_hbm.at[i >> 3].at[pl.ds(i & 7, 1), :],
                        out_v.at[b, pl.ds(grp * 16 + lane, 1), :],
                        gsems[b],
                    ).start()
            for k in range(CHUNK // 8):
                pltpu.make_async_copy(
                    table_hbm.at[0], out_v.at[b, pl.ds(k * 8, 8), :],
                    gsems[b]).wait()
            pltpu.make_async_copy(
                out_v.at[b], out_hbm.at[pl.ds(base + c * CHUNK, CHUNK)],
                wsems[b]).start()
        return carry

    lax.fori_loop(0, NCHUNK // 2, chunk_pair, 0)
    for b in range(2):
        pltpu.make_async_copy(
            out_v.at[b], out_hbm.at[pl.ds(0, CHUNK)], wsems[b]).wait()


def _sc_gather(word_table, idx3):
    mesh = plsc.VectorSubcoreMesh(core_axis_name="c", subcore_axis_name="s")
    return pl.kernel(
        _sc_gather_body,
        mesh=mesh,
        out_type=jax.ShapeDtypeStruct((R, EMB), jnp.float32),
        scratch_types=[
            pltpu.VMEM((NCHUNK, CHUNK), jnp.int32),
            pltpu.VMEM((2, CHUNK, EMB), jnp.float32),
            pltpu.SemaphoreType.DMA,
            pltpu.SemaphoreType.DMA,
            pltpu.SemaphoreType.DMA,
            pltpu.SemaphoreType.DMA,
        ],
    )(word_table, idx3)


def _tc_body(vis_ref, word_ref, seg_ref, pos_ref, segtab_ref, w_ref,
             g_ref, bta_ref, out_ref):
    vis = vis_ref[...].reshape(RB, PHOTO_DIM)
    acc = jnp.dot(vis, w_ref[...], preferred_element_type=jnp.float32)
    acc = acc.reshape(L, BB, EMB)
    word = word_ref[...]
    sg = seg_ref[...]
    t0 = segtab_ref[0:1, :].reshape(1, 1, EMB)
    t1 = segtab_ref[1:2, :].reshape(1, 1, EMB)
    t2 = segtab_ref[2:3, :].reshape(1, 1, EMB)
    seg_e = jnp.where(sg == 0.0, t0, jnp.where(sg == 1.0, t1, t2))
    emb = acc + word + pos_ref[...] + seg_e
    mean = jnp.mean(emb, axis=2, keepdims=True)
    d = emb - mean
    var = jnp.mean(d * d, axis=2, keepdims=True)
    out_ref[...] = (d * lax.rsqrt(var + 1e-6) * g_ref[...].reshape(1, 1, EMB)
                    + bta_ref[...].reshape(1, 1, EMB))


def _tc_fused(vis_t, word4, seg3, pos3, seg_table, w_vis, g2, b2,
              interpret=False):
    grid = (B // BB,)
    return pl.pallas_call(
        _tc_body,
        grid=grid,
        in_specs=[
            pl.BlockSpec((L, BB, PHOTO_DIM), lambda i: (0, i, 0)),
            pl.BlockSpec((L, BB, EMB), lambda i: (0, i, 0)),
            pl.BlockSpec((L, BB, 1), lambda i: (0, i, 0)),
            pl.BlockSpec((L, 1, EMB), lambda i: (0, 0, 0)),
            pl.BlockSpec((3, EMB), lambda i: (0, 0)),
            pl.BlockSpec((PHOTO_DIM, EMB), lambda i: (0, 0)),
            pl.BlockSpec((1, EMB), lambda i: (0, 0)),
            pl.BlockSpec((1, EMB), lambda i: (0, 0)),
        ],
        out_specs=pl.BlockSpec((L, BB, EMB), lambda i: (0, i, 0)),
        out_shape=jax.ShapeDtypeStruct((L, B, EMB), jnp.float32),
        compiler_params=pltpu.CompilerParams(
            dimension_semantics=("arbitrary",),
        ),
        interpret=interpret,
    )(vis_t, word4, seg3, pos3, seg_table, w_vis, g2, b2)


def kernel(visual, src, seg, word_table, pos_table, seg_table, W_vis,
           b_vis, ln_gamma, ln_beta):
    table3 = word_table.reshape(VOCAB // 8, 8, EMB)
    src_t = jnp.transpose(src, (1, 0))
    seg_t = jnp.transpose(seg, (1, 0))
    vis_t = jnp.transpose(visual, (1, 0, 2))
    idx3 = src_t.reshape(NW, NCHUNK, CHUNK)
    word2 = _sc_gather(table3, idx3)
    word4 = word2.reshape(L, B, EMB)
    seg3 = seg_t.astype(jnp.float32).reshape(L, B, 1)
    pos3 = (pos_table[:L] + b_vis).reshape(L, 1, EMB)
    g2 = ln_gamma.reshape(1, EMB)
    b2 = ln_beta.reshape(1, EMB)
    out_t = _tc_fused(vis_t, word4, seg3, pos3, seg_table, W_vis, g2, b2)
    return jnp.transpose(out_t, (1, 0, 2))


# speedup vs baseline: 2.4832x; 1.0135x over previous
import jax
import jax.numpy as jnp
from jax import lax
from jax.experimental import pallas as pl
from jax.experimental.pallas import tpu as pltpu
from jax.experimental.pallas import tpu_sc as plsc

VOCAB = 1000000
EMB = 64
PHOTO_DIM = 1024
B = 1024
L = 50

R = B * L
NC, NS = 2, 16
NW = NC * NS
ROWS_PER_W = R // NW
CHUNK = 32
NCHUNK = ROWS_PER_W // CHUNK

BB = 8
RB = BB * L


def _sc_gather_body(table_hbm, idx_hbm, out_hbm,
                    idx_v, out_v,
                    gsem0, gsem1, wsem0, wsem1):
    wid = lax.axis_index("s") * NC + lax.axis_index("c")
    base = wid * ROWS_PER_W
    pltpu.sync_copy(idx_hbm.at[wid], idx_v)
    gsems = (gsem0, gsem1)
    wsems = (wsem0, wsem1)

    def chunk_pair(g, carry):
        for b in range(2):
            c = 2 * g + b

            @pl.when(g > 0)
            def _():
                pltpu.make_async_copy(
                    out_v.at[b], out_hbm.at[pl.ds(0, CHUNK)],
                    wsems[b]).wait()

            for grp in range(CHUNK // 16):
                vec = idx_v[c, pl.ds(grp * 16, 16)]
                for lane in range(16):
                    i = vec[lane]
                    pltpu.make_async_copy(
                        table_hbm.at[i >> 3].at[pl.ds(i & 7, 1), :],
                        out_v.at[b, pl.ds(grp * 16 + lane, 1), :],
                        gsems[b],
                    ).start()
            for k in range(CHUNK // 8):
                pltpu.make_async_copy(
                    table_hbm.at[0], out_v.at[b, pl.ds(k * 8, 8), :],
                    gsems[b]).wait()
            pltpu.make_async_copy(
                out_v.at[b], out_hbm.at[pl.ds(base + c * CHUNK, CHUNK)],
                wsems[b]).start()
        return carry

    lax.fori_loop(0, NCHUNK // 2, chunk_pair, 0)
    for b in range(2):
        pltpu.make_async_copy(
            out_v.at[b], out_hbm.at[pl.ds(0, CHUNK)], wsems[b]).wait()


def _sc_gather(word_table, idx3):
    mesh = plsc.VectorSubcoreMesh(core_axis_name="c", subcore_axis_name="s")
    return pl.kernel(
        _sc_gather_body,
        mesh=mesh,
        out_type=jax.ShapeDtypeStruct((R, EMB), jnp.float32),
        scratch_types=[
            pltpu.VMEM((NCHUNK, CHUNK), jnp.int32),
            pltpu.VMEM((2, CHUNK, EMB), jnp.float32),
            pltpu.SemaphoreType.DMA,
            pltpu.SemaphoreType.DMA,
            pltpu.SemaphoreType.DMA,
            pltpu.SemaphoreType.DMA,
        ],
    )(word_table, idx3)


def _tc_body_a(vis_ref, seg_ref, pos_ref, segtab_ref, w_ref, out_ref):
    vis = vis_ref[...].reshape(RB, PHOTO_DIM)
    acc = jnp.dot(vis, w_ref[...], preferred_element_type=jnp.float32)
    acc = acc.reshape(L, BB, EMB)
    sg = seg_ref[...]
    t0 = segtab_ref[0:1, :].reshape(1, 1, EMB)
    t1 = segtab_ref[1:2, :].reshape(1, 1, EMB)
    t2 = segtab_ref[2:3, :].reshape(1, 1, EMB)
    seg_e = jnp.where(sg == 0.0, t0, jnp.where(sg == 1.0, t1, t2))
    out_ref[...] = acc + pos_ref[...] + seg_e


def _tc_part(vis_t, seg3, pos3, seg_table, w_vis, interpret=False):
    grid = (B // BB,)
    return pl.pallas_call(
        _tc_body_a,
        grid=grid,
        in_specs=[
            pl.BlockSpec((L, BB, PHOTO_DIM), lambda i: (0, i, 0)),
            pl.BlockSpec((L, BB, 1), lambda i: (0, i, 0)),
            pl.BlockSpec((L, 1, EMB), lambda i: (0, 0, 0)),
            pl.BlockSpec((3, EMB), lambda i: (0, 0)),
            pl.BlockSpec((PHOTO_DIM, EMB), lambda i: (0, 0)),
        ],
        out_specs=pl.BlockSpec((L, BB, EMB), lambda i: (0, i, 0)),
        out_shape=jax.ShapeDtypeStruct((L, B, EMB), jnp.float32),
        compiler_params=pltpu.CompilerParams(
            dimension_semantics=("arbitrary",),
        ),
        interpret=interpret,
    )(vis_t, seg3, pos3, seg_table, w_vis)


def _tc_body_b(part_ref, word_ref, g_ref, bta_ref, out_ref):
    emb = part_ref[...] + word_ref[...]
    mean = jnp.mean(emb, axis=2, keepdims=True)
    d = emb - mean
    var = jnp.mean(d * d, axis=2, keepdims=True)
    out_ref[...] = (d * lax.rsqrt(var + 1e-6) * g_ref[...].reshape(1, 1, EMB)
                    + bta_ref[...].reshape(1, 1, EMB))


def _tc_finish(part, word4, g2, b2, interpret=False):
    BB2 = 32
    grid = (B // BB2,)
    return pl.pallas_call(
        _tc_body_b,
        grid=grid,
        in_specs=[
            pl.BlockSpec((L, BB2, EMB), lambda i: (0, i, 0)),
            pl.BlockSpec((L, BB2, EMB), lambda i: (0, i, 0)),
            pl.BlockSpec((1, EMB), lambda i: (0, 0)),
            pl.BlockSpec((1, EMB), lambda i: (0, 0)),
        ],
        out_specs=pl.BlockSpec((L, BB2, EMB), lambda i: (0, i, 0)),
        out_shape=jax.ShapeDtypeStruct((L, B, EMB), jnp.float32),
        compiler_params=pltpu.CompilerParams(
            dimension_semantics=("arbitrary",),
        ),
        interpret=interpret,
    )(part, word4, g2, b2)


def kernel(visual, src, seg, word_table, pos_table, seg_table, W_vis,
           b_vis, ln_gamma, ln_beta):
    table3 = word_table.reshape(VOCAB // 8, 8, EMB)
    src_t = jnp.transpose(src, (1, 0))
    seg_t = jnp.transpose(seg, (1, 0))
    vis_t = jnp.transpose(visual, (1, 0, 2))
    idx3 = src_t.reshape(NW, NCHUNK, CHUNK)
    word2 = _sc_gather(table3, idx3)
    word4 = word2.reshape(L, B, EMB)
    seg3 = seg_t.astype(jnp.float32).reshape(L, B, 1)
    pos3 = (pos_table[:L] + b_vis).reshape(L, 1, EMB)
    g2 = ln_gamma.reshape(1, EMB)
    b2 = ln_beta.reshape(1, EMB)
    part = _tc_part(vis_t, seg3, pos3, seg_table, W_vis)
    out_t = _tc_finish(part, word4, g2, b2)
    return jnp.transpose(out_t, (1, 0, 2))
